# Initial kernel scaffold; baseline (speedup 1.0000x reference)
import jax
import jax.numpy as jnp
from jax.experimental import pallas as pl


def kernel(x, embedding):
    raise NotImplementedError("write your pallas kernel here")


# speedup vs baseline: 1.1793x; 1.1793x over previous
import jax
import jax.numpy as jnp
from jax import lax
from jax.experimental import pallas as pl
from jax.experimental.pallas import tpu as pltpu

_N_EMB = 8192
_DIM = 256
_BETA = 0.25
_ROWS = 16384
_BLK = 256
_GRID = _ROWS // _BLK


def _vq_body(x_ref, emb_ref, idx_ref, q_ref, enc_ref, loss_ref,
             perp_ref, counts_ref, sse_ref):
    i = pl.program_id(0)

    @pl.when(i == 0)
    def _init():
        counts_ref[...] = jnp.zeros((1, _N_EMB), jnp.float32)
        sse_ref[0] = 0.0

    x = x_ref[...]
    emb = emb_ref[...]
    idx = idx_ref[...]

    enc = (lax.broadcasted_iota(jnp.int32, (_BLK, _N_EMB), 1)
           == idx).astype(jnp.float32)
    enc_ref[...] = enc
    counts_ref[...] += jnp.sum(enc, axis=0, keepdims=True)

    q = lax.dot_general(enc.astype(jnp.bfloat16), emb.astype(jnp.bfloat16),
                        (((1,), (0,)), ((), ())),
                        preferred_element_type=jnp.float32)
    q_ref[...] = x + (q - x)
    diff = q - x
    sse_ref[0] += jnp.sum(diff * diff)

    @pl.when(i == _GRID - 1)
    def _fin():
        loss_ref[0, 0] = _BETA * sse_ref[0] / (_ROWS * _DIM)
        avg = counts_ref[...] / _ROWS
        perp_ref[0, 0] = jnp.exp(-jnp.sum(avg * jnp.log(avg + 1e-10)))


@jax.jit
def kernel(x, embedding):
    xp = jnp.transpose(x, (0, 2, 3, 1))
    flat_x = xp.reshape(_ROWS, _DIM)

    distances = (jnp.sum(flat_x ** 2, axis=1, keepdims=True)
                 + jnp.sum(embedding ** 2, axis=1)
                 - 2.0 * jnp.matmul(flat_x, embedding.T))
    idx = jnp.argmin(distances, axis=1).astype(jnp.int32)[:, None]

    q, enc, loss, perp = pl.pallas_call(
        _vq_body,
        grid=(_GRID,),
        in_specs=[
            pl.BlockSpec((_BLK, _DIM), lambda i: (i, 0)),
            pl.BlockSpec((_N_EMB, _DIM), lambda i: (0, 0)),
            pl.BlockSpec((_BLK, 1), lambda i: (i, 0)),
        ],
        out_specs=[
            pl.BlockSpec((_BLK, _DIM), lambda i: (i, 0)),
            pl.BlockSpec((_BLK, _N_EMB), lambda i: (i, 0)),
            pl.BlockSpec(memory_space=pltpu.SMEM),
            pl.BlockSpec(memory_space=pltpu.SMEM),
        ],
        out_shape=[
            jax.ShapeDtypeStruct((_ROWS, _DIM), jnp.float32),
            jax.ShapeDtypeStruct((_ROWS, _N_EMB), jnp.float32),
            jax.ShapeDtypeStruct((1, 1), jnp.float32),
            jax.ShapeDtypeStruct((1, 1), jnp.float32),
        ],
        scratch_shapes=[
            pltpu.VMEM((1, _N_EMB), jnp.float32),
            pltpu.SMEM((1,), jnp.float32),
        ],
    )(flat_x, embedding, idx)

    quantized_out = jnp.transpose(q.reshape(16, 32, 32, _DIM), (0, 3, 1, 2))
    return (loss[0, 0], quantized_out, perp[0, 0], enc)
